# baseline (speedup 1.0000x reference)
import jax
import jax.numpy as jnp
from jax.experimental import pallas as pl
from jax.experimental.pallas import tpu as pltpu

_EPS = 1e-08
_B, _S, _D = 4, 2048, 768
_NUM_CLASSES_SEEN = 10
_TOP_K = 3
_N_DESC = 5
_S_CHUNK = 256
_NSTEPS = _S // _S_CHUNK


def _fused_body(x_ref, ck_ref, desc_ref, w_ref, g_ref, b_ref, t_ref,
                out_ref, acc_ref, dp_ref, ckn_ref, sim_ref):
    i = pl.program_id(0)

    partial = jnp.sum(x_ref[...], axis=1)

    @pl.when(i == 0)
    def _init():
        acc_ref[0:_B, :] = partial
        dp_ref[0:_N_DESC, :] = jax.lax.dot_general(
            desc_ref[...], w_ref[...], (((1,), (1,)), ((), ())),
            preferred_element_type=jnp.float32)
        ck = ck_ref[0:_NUM_CLASSES_SEEN, :]
        cknorm = jnp.sqrt(jnp.sum(ck * ck, axis=1, keepdims=True))
        ckn_ref[0:_NUM_CLASSES_SEEN, :] = ck / jnp.maximum(cknorm, _EPS)

    @pl.when(i > 0)
    def _accum():
        acc_ref[0:_B, :] = acc_ref[0:_B, :] + partial

    ps = jax.lax.dot_general(
        partial, ckn_ref[0:_NUM_CLASSES_SEEN, :], (((1,), (1,)), ((), ())),
        preferred_element_type=jnp.float32)

    @pl.when(i == 0)
    def _sinit():
        sim_ref[0:_B, 0:_NUM_CLASSES_SEEN] = ps

    @pl.when(i > 0)
    def _saccum():
        sim_ref[0:_B, 0:_NUM_CLASSES_SEEN] = (
            sim_ref[0:_B, 0:_NUM_CLASSES_SEEN] + ps)

    @pl.when(i == _NSTEPS - 1)
    def _epilogue():
        acc = acc_ref[0:_B, :]
        anorm = jnp.sqrt(jnp.sum(acc * acc, axis=1, keepdims=True))
        sims = (sim_ref[0:_B, 0:_NUM_CLASSES_SEEN]
                / jnp.maximum(anorm, _S * _EPS))

        t = t_ref[0]

        col = jax.lax.broadcasted_iota(jnp.int32, (_B, _NUM_CLASSES_SEEN), 1)
        s = sims
        vals = []
        idxs = []
        for _ in range(_TOP_K):
            m = jnp.max(s, axis=1, keepdims=True)
            idx = jnp.min(jnp.where(s >= m, col, _NUM_CLASSES_SEEN + 1),
                          axis=1, keepdims=True)
            vals.append(m)
            idxs.append(idx)
            s = jnp.where(col == idx, -jnp.inf, s)

        exps = [jnp.exp((v - vals[0]) * t) for v in vals]
        denom = exps[0] + exps[1] + exps[2]
        ws = [e / denom for e in exps]

        dcol = jax.lax.broadcasted_iota(jnp.int32, (_B, _N_DESC), 1)
        dw = jnp.zeros((_B, _N_DESC), jnp.float32)
        for k in range(_TOP_K):
            didx = jax.lax.rem(idxs[k], _N_DESC)
            dw = dw + jnp.where(dcol == didx, ws[k], 0.0)

        proj = jax.lax.dot_general(
            dw, dp_ref[0:_N_DESC, :], (((1,), (0,)), ((), ())),
            preferred_element_type=jnp.float32)

        mu = jnp.mean(proj, axis=1, keepdims=True)
        ctr = proj - mu
        var = jnp.mean(ctr * ctr, axis=1, keepdims=True)
        ln = (ctr * jax.lax.rsqrt(var + 1e-05) * g_ref[...].reshape(1, _D)
              + b_ref[...].reshape(1, _D))

        out_ref[:, 0, :] = ln


@jax.jit
def kernel(x_embed, prompt_key, task_key, desc_emb, W_proj, ln_gamma,
           ln_beta, temperature):
    del task_key

    out = pl.pallas_call(
        _fused_body,
        grid=(_NSTEPS,),
        in_specs=[
            pl.BlockSpec((_B, _S_CHUNK, _D), lambda i: (0, i, 0)),
            pl.BlockSpec((16, _D), lambda i: (0, 0)),
            pl.BlockSpec((_N_DESC, _D), lambda i: (0, 0)),
            pl.BlockSpec((_D, _D), lambda i: (0, 0)),
            pl.BlockSpec((_D,), lambda i: (0,)),
            pl.BlockSpec((_D,), lambda i: (0,)),
            pl.BlockSpec(memory_space=pltpu.SMEM),
        ],
        out_specs=pl.BlockSpec((_B, 1, _D), lambda i: (0, 0, 0)),
        out_shape=jax.ShapeDtypeStruct((_B, 1, _D), jnp.float32),
        scratch_shapes=[pltpu.VMEM((8, _D), jnp.float32),
                        pltpu.VMEM((8, _D), jnp.float32),
                        pltpu.VMEM((16, _D), jnp.float32),
                        pltpu.VMEM((8, 128), jnp.float32)],
    )(x_embed, prompt_key, desc_emb, W_proj, ln_gamma, ln_beta, temperature)
    return out


# speedup vs baseline: 1.1239x; 1.1239x over previous
import jax
import jax.numpy as jnp
from jax.experimental import pallas as pl
from jax.experimental.pallas import tpu as pltpu

_EPS = 1e-08
_B, _S, _D = 4, 2048, 768
_NUM_CLASSES_SEEN = 10
_TOP_K = 3
_N_DESC = 5
_S_CHUNK = 512
_NSTEPS = _S // _S_CHUNK


def _fused_body(x_ref, ck_ref, desc_ref, w_ref, g_ref, b_ref, t_ref,
                out_ref, acc_ref, dp_ref):
    i = pl.program_id(0)

    partial = jnp.sum(x_ref[...], axis=1)

    @pl.when(i == 0)
    def _init():
        acc_ref[0:_B, :] = partial
        dp_ref[0:_N_DESC, :] = jax.lax.dot_general(
            desc_ref[...], w_ref[...], (((1,), (1,)), ((), ())),
            preferred_element_type=jnp.float32)

    @pl.when(i > 0)
    def _accum():
        acc_ref[0:_B, :] = acc_ref[0:_B, :] + partial

    @pl.when(i == _NSTEPS - 1)
    def _epilogue():
        mean = acc_ref[0:_B, :] * (1.0 / _S)
        xnorm = jnp.sqrt(jnp.sum(mean * mean, axis=1, keepdims=True))
        xn = mean / jnp.maximum(xnorm, _EPS)

        ck = ck_ref[0:_NUM_CLASSES_SEEN, :]
        cknorm = jnp.sqrt(jnp.sum(ck * ck, axis=1, keepdims=True))
        ckn = ck / jnp.maximum(cknorm, _EPS)

        sims = jax.lax.dot_general(
            xn, ckn, (((1,), (1,)), ((), ())),
            preferred_element_type=jnp.float32)

        t = t_ref[0]

        col = jax.lax.broadcasted_iota(jnp.int32, (_B, _NUM_CLASSES_SEEN), 1)
        s = sims
        vals = []
        idxs = []
        for _ in range(_TOP_K):
            m = jnp.max(s, axis=1, keepdims=True)
            idx = jnp.min(jnp.where(s >= m, col, _NUM_CLASSES_SEEN + 1),
                          axis=1, keepdims=True)
            vals.append(m)
            idxs.append(idx)
            s = jnp.where(col == idx, -jnp.inf, s)

        exps = [jnp.exp((v - vals[0]) * t) for v in vals]
        denom = exps[0] + exps[1] + exps[2]
        ws = [e / denom for e in exps]

        dcol = jax.lax.broadcasted_iota(jnp.int32, (_B, _N_DESC), 1)
        dw = jnp.zeros((_B, _N_DESC), jnp.float32)
        for k in range(_TOP_K):
            didx = jax.lax.rem(idxs[k], _N_DESC)
            dw = dw + jnp.where(dcol == didx, ws[k], 0.0)

        proj = jax.lax.dot_general(
            dw, dp_ref[0:_N_DESC, :], (((1,), (0,)), ((), ())),
            preferred_element_type=jnp.float32)

        mu = jnp.mean(proj, axis=1, keepdims=True)
        ctr = proj - mu
        var = jnp.mean(ctr * ctr, axis=1, keepdims=True)
        ln = (ctr * jax.lax.rsqrt(var + 1e-05) * g_ref[...].reshape(1, _D)
              + b_ref[...].reshape(1, _D))

        out_ref[:, 0, :] = ln


@jax.jit
def kernel(x_embed, prompt_key, task_key, desc_emb, W_proj, ln_gamma,
           ln_beta, temperature):
    del task_key

    out = pl.pallas_call(
        _fused_body,
        grid=(_NSTEPS,),
        in_specs=[
            pl.BlockSpec((_B, _S_CHUNK, _D), lambda i: (0, i, 0)),
            pl.BlockSpec((16, _D), lambda i: (0, 0)),
            pl.BlockSpec((_N_DESC, _D), lambda i: (0, 0)),
            pl.BlockSpec((_D, _D), lambda i: (0, 0)),
            pl.BlockSpec((_D,), lambda i: (0,)),
            pl.BlockSpec((_D,), lambda i: (0,)),
            pl.BlockSpec(memory_space=pltpu.SMEM),
        ],
        out_specs=pl.BlockSpec((_B, 1, _D), lambda i: (0, 0, 0)),
        out_shape=jax.ShapeDtypeStruct((_B, 1, _D), jnp.float32),
        scratch_shapes=[pltpu.VMEM((8, _D), jnp.float32),
                        pltpu.VMEM((8, _D), jnp.float32)],
    )(x_embed, prompt_key, desc_emb, W_proj, ln_gamma, ln_beta, temperature)
    return out


# speedup vs baseline: 1.1355x; 1.0104x over previous
import jax
import jax.numpy as jnp
from jax.experimental import pallas as pl
from jax.experimental.pallas import tpu as pltpu

_EPS = 1e-08
_B, _S, _D = 4, 2048, 768
_NUM_CLASSES_SEEN = 10
_TOP_K = 3
_N_DESC = 5
_S_CHUNK = 1024
_NSTEPS = _S // _S_CHUNK


def _fused_body(x_ref, ck_ref, desc_ref, w_ref, g_ref, b_ref, t_ref,
                out_ref, acc_ref, dp_ref):
    i = pl.program_id(0)

    partial = jnp.sum(x_ref[...], axis=1)

    @pl.when(i == 0)
    def _init():
        acc_ref[0:_B, :] = partial
        dp_ref[0:_N_DESC, :] = jax.lax.dot_general(
            desc_ref[...], w_ref[...], (((1,), (1,)), ((), ())),
            preferred_element_type=jnp.float32)

    @pl.when(i > 0)
    def _accum():
        acc_ref[0:_B, :] = acc_ref[0:_B, :] + partial

    @pl.when(i == _NSTEPS - 1)
    def _epilogue():
        mean = acc_ref[0:_B, :] * (1.0 / _S)
        xnorm = jnp.sqrt(jnp.sum(mean * mean, axis=1, keepdims=True))
        xn = mean / jnp.maximum(xnorm, _EPS)

        ck = ck_ref[0:_NUM_CLASSES_SEEN, :]
        cknorm = jnp.sqrt(jnp.sum(ck * ck, axis=1, keepdims=True))
        ckn = ck / jnp.maximum(cknorm, _EPS)

        sims = jax.lax.dot_general(
            xn, ckn, (((1,), (1,)), ((), ())),
            preferred_element_type=jnp.float32)

        t = t_ref[0]

        col = jax.lax.broadcasted_iota(jnp.int32, (_B, _NUM_CLASSES_SEEN), 1)
        s = sims
        vals = []
        idxs = []
        for _ in range(_TOP_K):
            m = jnp.max(s, axis=1, keepdims=True)
            idx = jnp.min(jnp.where(s >= m, col, _NUM_CLASSES_SEEN + 1),
                          axis=1, keepdims=True)
            vals.append(m)
            idxs.append(idx)
            s = jnp.where(col == idx, -jnp.inf, s)

        exps = [jnp.exp((v - vals[0]) * t) for v in vals]
        denom = exps[0] + exps[1] + exps[2]
        ws = [e / denom for e in exps]

        dcol = jax.lax.broadcasted_iota(jnp.int32, (_B, _N_DESC), 1)
        dw = jnp.zeros((_B, _N_DESC), jnp.float32)
        for k in range(_TOP_K):
            didx = jax.lax.rem(idxs[k], _N_DESC)
            dw = dw + jnp.where(dcol == didx, ws[k], 0.0)

        proj = jax.lax.dot_general(
            dw, dp_ref[0:_N_DESC, :], (((1,), (0,)), ((), ())),
            preferred_element_type=jnp.float32)

        mu = jnp.mean(proj, axis=1, keepdims=True)
        ctr = proj - mu
        var = jnp.mean(ctr * ctr, axis=1, keepdims=True)
        ln = (ctr * jax.lax.rsqrt(var + 1e-05) * g_ref[...].reshape(1, _D)
              + b_ref[...].reshape(1, _D))

        out_ref[:, 0, :] = ln


@jax.jit
def kernel(x_embed, prompt_key, task_key, desc_emb, W_proj, ln_gamma,
           ln_beta, temperature):
    del task_key

    out = pl.pallas_call(
        _fused_body,
        grid=(_NSTEPS,),
        in_specs=[
            pl.BlockSpec((_B, _S_CHUNK, _D), lambda i: (0, i, 0)),
            pl.BlockSpec((16, _D), lambda i: (0, 0)),
            pl.BlockSpec((_N_DESC, _D), lambda i: (0, 0)),
            pl.BlockSpec((_D, _D), lambda i: (0, 0)),
            pl.BlockSpec((_D,), lambda i: (0,)),
            pl.BlockSpec((_D,), lambda i: (0,)),
            pl.BlockSpec(memory_space=pltpu.SMEM),
        ],
        out_specs=pl.BlockSpec((_B, 1, _D), lambda i: (0, 0, 0)),
        out_shape=jax.ShapeDtypeStruct((_B, 1, _D), jnp.float32),
        scratch_shapes=[pltpu.VMEM((8, _D), jnp.float32),
                        pltpu.VMEM((8, _D), jnp.float32)],
    )(x_embed, prompt_key, desc_emb, W_proj, ln_gamma, ln_beta, temperature)
    return out
